# baseline (speedup 1.0000x reference)
import jax
import jax.numpy as jnp
from jax import lax
from jax.experimental import pallas as pl

_S = 64.0
_MARGIN = 1.7

_ROWS = 1024
_C_BLOCK = 3584
_LANES = 128


def _acos_poly(x):
    p7 = -0.0012624911
    p6 = 0.0066700901
    p5 = -0.0170881256
    p4 = 0.0308918810
    p3 = -0.0501743046
    p2 = 0.0889789874
    p1 = -0.2145988016
    p0 = 1.5707963050
    r = p7
    for c in (p6, p5, p4, p3, p2, p1, p0):
        r = r * x + c
    return r * jnp.sqrt(jnp.maximum(1.0 - x, 0.0))


def _tc_body(lab_ref, x_ref, o_ref):
    j = pl.program_id(0)
    lab = lab_ref[0, 0, :]
    local = (lab - j * _C_BLOCK)[:, None]
    col = lax.broadcasted_iota(jnp.int32, (_ROWS, _LANES), 1)
    acc = jnp.zeros((_ROWS, _LANES), jnp.float32)
    for k in range(_C_BLOCK // _LANES):
        xk = x_ref[:, k * _LANES:(k + 1) * _LANES]
        acc = acc + jnp.where(col + (k * _LANES) == local, xk, 0.0)
    t = jnp.sum(acc, axis=1)
    m = _S * jnp.cos(_MARGIN * _acos_poly(t))
    mb = m[:, None]
    for k in range(_C_BLOCK // _LANES):
        xk = x_ref[:, k * _LANES:(k + 1) * _LANES]
        o_ref[:, k * _LANES:(k + 1) * _LANES] = jnp.where(
            col + (k * _LANES) == local, mb, _S * xk)


def kernel(logits, labels, embeddings):
    del embeddings
    rows, cols = logits.shape
    labels = labels.astype(jnp.int32)
    n_c = pl.cdiv(cols, _C_BLOCK)
    lab3 = labels.reshape(1, 1, rows)
    return pl.pallas_call(
        _tc_body,
        grid=(n_c,),
        in_specs=[
            pl.BlockSpec((1, 1, rows), lambda j: (0, 0, 0)),
            pl.BlockSpec((rows, _C_BLOCK), lambda j: (0, j)),
        ],
        out_specs=pl.BlockSpec((rows, _C_BLOCK), lambda j: (0, j)),
        out_shape=jax.ShapeDtypeStruct((rows, cols), jnp.float32),
    )(lab3, logits)


# speedup vs baseline: 1.0071x; 1.0071x over previous
import jax
import jax.numpy as jnp
from jax import lax
from jax.experimental import pallas as pl

_S = 64.0
_MARGIN = 1.7

_ROWS = 1024
_C_BLOCK = 2560
_LANES = 128


def _acos_poly(x):
    p7 = -0.0012624911
    p6 = 0.0066700901
    p5 = -0.0170881256
    p4 = 0.0308918810
    p3 = -0.0501743046
    p2 = 0.0889789874
    p1 = -0.2145988016
    p0 = 1.5707963050
    r = p7
    for c in (p6, p5, p4, p3, p2, p1, p0):
        r = r * x + c
    return r * jnp.sqrt(jnp.maximum(1.0 - x, 0.0))


def _tc_body(lab_ref, x_ref, o_ref):
    j = pl.program_id(0)
    lab = lab_ref[0, 0, :]
    local = (lab - j * _C_BLOCK)[:, None]
    lane = local & (_LANES - 1)
    chunk = local >> 7
    n_k = _C_BLOCK // _LANES

    acc = x_ref[:, 0:_LANES]
    for k in range(1, n_k):
        xk = x_ref[:, k * _LANES:(k + 1) * _LANES]
        acc = jnp.where(chunk == k, xk, acc)

    t = jnp.take_along_axis(acc, lane, axis=1)[:, 0]
    m = _S * jnp.cos(_MARGIN * _acos_poly(t))
    col = lax.broadcasted_iota(jnp.int32, (_ROWS, _LANES), 1)
    patched = jnp.where(col == lane, m[:, None], _S * acc)

    for k in range(n_k):
        xk = x_ref[:, k * _LANES:(k + 1) * _LANES]
        o_ref[:, k * _LANES:(k + 1) * _LANES] = jnp.where(
            chunk == k, patched, _S * xk)


def kernel(logits, labels, embeddings):
    del embeddings
    rows, cols = logits.shape
    labels = labels.astype(jnp.int32)
    n_c = pl.cdiv(cols, _C_BLOCK)
    lab3 = labels.reshape(1, 1, rows)
    return pl.pallas_call(
        _tc_body,
        grid=(n_c,),
        in_specs=[
            pl.BlockSpec((1, 1, rows), lambda j: (0, 0, 0)),
            pl.BlockSpec((rows, _C_BLOCK), lambda j: (0, j)),
        ],
        out_specs=pl.BlockSpec((rows, _C_BLOCK), lambda j: (0, j)),
        out_shape=jax.ShapeDtypeStruct((rows, cols), jnp.float32),
    )(lab3, logits)


# speedup vs baseline: 1.0085x; 1.0014x over previous
import jax
import jax.numpy as jnp
from jax import lax
from jax.experimental import pallas as pl

_S = 64.0
_MARGIN = 1.7

_ROWS = 1024
_C_BLOCK = 3584
_LANES = 128


def _acos_poly(x):
    p7 = -0.0012624911
    p6 = 0.0066700901
    p5 = -0.0170881256
    p4 = 0.0308918810
    p3 = -0.0501743046
    p2 = 0.0889789874
    p1 = -0.2145988016
    p0 = 1.5707963050
    r = p7
    for c in (p6, p5, p4, p3, p2, p1, p0):
        r = r * x + c
    return r * jnp.sqrt(jnp.maximum(1.0 - x, 0.0))


def _tc_body(lab_ref, x_ref, o_ref):
    j = pl.program_id(0)
    lab = lab_ref[0, 0, :]
    local = (lab - j * _C_BLOCK)[:, None]
    lane = local & (_LANES - 1)
    chunk = local >> 7
    n_k = _C_BLOCK // _LANES

    acc = x_ref[:, 0:_LANES]
    for k in range(1, n_k):
        xk = x_ref[:, k * _LANES:(k + 1) * _LANES]
        acc = jnp.where(chunk == k, xk, acc)

    t = jnp.take_along_axis(acc, lane, axis=1)[:, 0]
    m = _S * jnp.cos(_MARGIN * _acos_poly(t))
    col = lax.broadcasted_iota(jnp.int32, (_ROWS, _LANES), 1)
    patched = jnp.where(col == lane, m[:, None], _S * acc)

    for k in range(n_k):
        xk = x_ref[:, k * _LANES:(k + 1) * _LANES]
        o_ref[:, k * _LANES:(k + 1) * _LANES] = jnp.where(
            chunk == k, patched, _S * xk)


def kernel(logits, labels, embeddings):
    del embeddings
    rows, cols = logits.shape
    labels = labels.astype(jnp.int32)
    n_c = pl.cdiv(cols, _C_BLOCK)
    lab3 = labels.reshape(1, 1, rows)
    return pl.pallas_call(
        _tc_body,
        grid=(n_c,),
        in_specs=[
            pl.BlockSpec((1, 1, rows), lambda j: (0, 0, 0)),
            pl.BlockSpec((rows, _C_BLOCK), lambda j: (0, j)),
        ],
        out_specs=pl.BlockSpec((rows, _C_BLOCK), lambda j: (0, j)),
        out_shape=jax.ShapeDtypeStruct((rows, cols), jnp.float32),
    )(lab3, logits)
